# baseline (speedup 1.0000x reference)
import functools

import jax
import jax.numpy as jnp
from jax import lax
from jax.experimental import pallas as pl
from jax.experimental.pallas import tpu as pltpu
from jax.experimental.pallas import tpu_sc as plsc


def _make_sc_gather(d: int, b: int):
    info = plsc.get_sparse_core_info()
    nw = info.num_cores * info.num_subcores
    b_per_w = b // nw
    mesh = plsc.VectorSubcoreMesh(core_axis_name="c", subcore_axis_name="s")

    @functools.partial(
        pl.kernel,
        mesh=mesh,
        out_type=jax.ShapeDtypeStruct((b, d), jnp.float32),
        scratch_types=[
            pltpu.VMEM((b_per_w,), jnp.int32),
            pltpu.VMEM((b_per_w, d), jnp.float32),
            pltpu.SemaphoreType.DMA,
        ],
    )
    def gather_kernel(table_hbm, idx_hbm, out_hbm, idx_v, rows_v, sem):
        wid = lax.axis_index("s") * info.num_cores + lax.axis_index("c")
        base = wid * b_per_w
        pltpu.sync_copy(idx_hbm.at[pl.ds(base, b_per_w)], idx_v)
        pltpu.async_copy(table_hbm.at[idx_v], rows_v, sem).wait()
        pltpu.sync_copy(rows_v, out_hbm.at[pl.ds(base, b_per_w)])

    return gather_kernel


def _score_body(head_ref, rel_ref, tail_ref, out_ref):
    hr = head_ref[...] * rel_ref[...]
    out_ref[...] = jnp.sum(tail_ref[...] * hr[:, None, :], axis=2)


def kernel(head_embs, tail_embs, rel_idx, w_relation):
    b, n_neg, d = tail_embs.shape

    rel = jnp.take(w_relation, rel_idx, axis=0)

    blk = 128
    grid = (b // blk,)
    score = pl.pallas_call(
        _score_body,
        grid=grid,
        in_specs=[
            pl.BlockSpec((blk, d), lambda i: (i, 0)),
            pl.BlockSpec((blk, d), lambda i: (i, 0)),
            pl.BlockSpec((blk, n_neg, d), lambda i: (i, 0, 0)),
        ],
        out_specs=pl.BlockSpec((blk, n_neg), lambda i: (i, 0)),
        out_shape=jax.ShapeDtypeStruct((b, n_neg), jnp.float32),
    )(head_embs, rel, tail_embs)
    return score


# speedup vs baseline: 1.1611x; 1.1611x over previous
import functools

import jax
import jax.numpy as jnp
from jax import lax
from jax.experimental import pallas as pl
from jax.experimental.pallas import tpu as pltpu
from jax.experimental.pallas import tpu_sc as plsc


def _make_sc_gather(d: int, b: int):
    info = plsc.get_sparse_core_info()
    nw = info.num_cores * info.num_subcores
    b_per_w = b // nw
    mesh = plsc.VectorSubcoreMesh(core_axis_name="c", subcore_axis_name="s")

    @functools.partial(
        pl.kernel,
        mesh=mesh,
        out_type=jax.ShapeDtypeStruct((b, d), jnp.float32),
        scratch_types=[
            pltpu.VMEM((b_per_w,), jnp.int32),
            pltpu.VMEM((b_per_w, d), jnp.float32),
            pltpu.SemaphoreType.DMA,
        ],
    )
    def gather_kernel(table_hbm, idx_hbm, out_hbm, idx_v, rows_v, sem):
        wid = lax.axis_index("s") * info.num_cores + lax.axis_index("c")
        base = wid * b_per_w
        pltpu.sync_copy(idx_hbm.at[pl.ds(base, b_per_w)], idx_v)
        pltpu.async_copy(table_hbm.at[idx_v], rows_v, sem).wait()
        pltpu.sync_copy(rows_v, out_hbm.at[pl.ds(base, b_per_w)])

    return gather_kernel


def _score_body(head_ref, rel_ref, tail_ref, out_ref):
    hr = head_ref[...] * rel_ref[...]
    out_ref[...] = jnp.sum(tail_ref[...] * hr[:, None, :], axis=2)


def kernel(head_embs, tail_embs, rel_idx, w_relation):
    b, n_neg, d = tail_embs.shape

    rel = jnp.take(w_relation, rel_idx, axis=0)

    blk = 512
    grid = (b // blk,)
    score = pl.pallas_call(
        _score_body,
        grid=grid,
        in_specs=[
            pl.BlockSpec((blk, d), lambda i: (i, 0)),
            pl.BlockSpec((blk, d), lambda i: (i, 0)),
            pl.BlockSpec((blk, n_neg, d), lambda i: (i, 0, 0)),
        ],
        out_specs=pl.BlockSpec((blk, n_neg), lambda i: (i, 0)),
        out_shape=jax.ShapeDtypeStruct((b, n_neg), jnp.float32),
    )(head_embs, rel, tail_embs)
    return score
